# Initial kernel scaffold; baseline (speedup 1.0000x reference)
import jax
import jax.numpy as jnp
from jax.experimental import pallas as pl


def kernel(x, shared_down, shared_up, adapters_down, adapters_up, gates):
    raise NotImplementedError("write your pallas kernel here")


# speedup vs baseline: 19.8996x; 19.8996x over previous
import functools

import jax
import jax.numpy as jnp
from jax import lax
from jax.experimental import pallas as pl
from jax.experimental.pallas import tpu as pltpu

L = 20
H = 2048
R = 512
AR = 64
E = 256
TOPK = 8
B = 32
K = 8
N = B * K


_INV_SQRT2 = 0.7071067811865476


def _erf_gelu(v):
    return v * (0.5 * (1.0 + lax.erf(v * _INV_SQRT2)))


def _body(x_ref, sd_ref, su_ref, ad_ref, au_ref, g_ref,
          logits_ref, idx_ref, shared_ref):
    l = pl.program_id(0)

    @pl.when(l == 0)
    def _():
        s = _erf_gelu(lax.dot_general(
            x_ref[...], sd_ref[...], (((1,), (1,)), ((), ()))))
        shared_ref[...] = lax.dot_general(
            s, su_ref[...], (((1,), (1,)), ((), ())))

    a = _erf_gelu(lax.dot_general(
        x_ref[...], ad_ref[0], (((1,), (1,)), ((), ()))))
    adapter = lax.dot_general(a, au_ref[0], (((1,), (1,)), ((), ())))
    h = shared_ref[...] + adapter
    logits = lax.dot_general(h, g_ref[0], (((1,), (1,)), ((), ())))
    logits_ref[0] = logits

    col = lax.broadcasted_iota(jnp.int32, (N, E), 1)
    work = logits
    cols = []
    for _ in range(TOPK):
        m = jnp.max(work, axis=1, keepdims=True)
        idxj = jnp.min(jnp.where(work == m, col, E), axis=1, keepdims=True)
        cols.append(idxj)
        work = jnp.where(col == idxj, -jnp.inf, work)
    idx_ref[0] = jnp.concatenate(cols, axis=1)


@jax.jit
def kernel(x, shared_down, shared_up, adapters_down, adapters_up, gates):
    xf = x.reshape(N, H)
    logits, idx = pl.pallas_call(
        _body,
        grid=(L,),
        in_specs=[
            pl.BlockSpec((N, H), lambda l: (0, 0)),
            pl.BlockSpec((R, H), lambda l: (0, 0)),
            pl.BlockSpec((H, R), lambda l: (0, 0)),
            pl.BlockSpec((1, AR, H), lambda l: (l, 0, 0)),
            pl.BlockSpec((1, H, AR), lambda l: (l, 0, 0)),
            pl.BlockSpec((1, E, H), lambda l: (l, 0, 0)),
        ],
        out_specs=[
            pl.BlockSpec((1, N, E), lambda l: (l, 0, 0)),
            pl.BlockSpec((1, N, TOPK), lambda l: (l, 0, 0)),
        ],
        out_shape=[
            jax.ShapeDtypeStruct((L, N, E), jnp.float32),
            jax.ShapeDtypeStruct((L, N, TOPK), jnp.int32),
        ],
        scratch_shapes=[pltpu.VMEM((N, H), jnp.float32)],
    )(xf, shared_down, shared_up, adapters_down, adapters_up, gates)
    return (idx.reshape(L, B, K, TOPK), logits.reshape(L, B, K, E))
